# baseline (speedup 1.0000x reference)
import functools

import jax
import jax.numpy as jnp
from jax import lax
from jax.experimental import pallas as pl
from jax.experimental.pallas import tpu as pltpu

try:
    from jax.experimental.pallas import tpu_sc as plsc
except ImportError:
    plsc = None

_TOKENS = 8192
_HIDDEN = 2048
_EXPERTS = 64
_K = 8


def _matmul_body(x_ref, wt_ref, b_ref, logits_ref):
    logits_ref[...] = (
        jnp.dot(x_ref[...], wt_ref[...], preferred_element_type=jnp.float32)
        + b_ref[...]
    )


def _tc_logits(x, wt, b, bt=1024):
    return pl.pallas_call(
        _matmul_body,
        grid=(_TOKENS // bt,),
        in_specs=[
            pl.BlockSpec((bt, _HIDDEN), lambda i: (i, 0)),
            pl.BlockSpec((_HIDDEN, _EXPERTS), lambda i: (0, 0)),
            pl.BlockSpec((1, _EXPERTS), lambda i: (0, 0)),
        ],
        out_specs=pl.BlockSpec((bt, _EXPERTS), lambda i: (i, 0)),
        out_shape=jax.ShapeDtypeStruct((_TOKENS, _EXPERTS), jnp.float32),
        compiler_params=pltpu.CompilerParams(
            dimension_semantics=("arbitrary",),
        ),
    )(x, wt, b)


_GDN = lax.GatherDimensionNumbers(
    offset_dims=(), collapsed_slice_dims=(0,), start_index_map=(0,))


def _lane_shuffle(x, perm):
    return lax.gather(
        x, perm[:, None], _GDN, (1,),
        mode=lax.GatherScatterMode.PROMISE_IN_BOUNDS)


def _bfly(x, op):
    idx = lax.iota(jnp.int32, 16)
    for sh in (8, 4, 2, 1):
        perm = jnp.bitwise_xor(idx, sh)
        x = op(x, _lane_shuffle(x, perm))
    return x


def _make_sc_router():
    info = plsc.get_sparse_core_info()
    nc, ns, nl = info.num_cores, info.num_subcores, info.num_lanes
    nw = nc * ns
    rows_per_w = _TOKENS // nw
    nv = _EXPERTS // nl
    mesh = plsc.VectorSubcoreMesh(core_axis_name="c", subcore_axis_name="s")

    @functools.partial(
        pl.kernel,
        mesh=mesh,
        out_type=[
            jax.ShapeDtypeStruct((_TOKENS, _EXPERTS), jnp.float32),
            jax.ShapeDtypeStruct((_TOKENS, nl), jnp.int32),
        ],
        scratch_types=[
            pltpu.VMEM((rows_per_w, _EXPERTS), jnp.float32),
            pltpu.VMEM((rows_per_w, _EXPERTS), jnp.float32),
            pltpu.VMEM((rows_per_w, nl), jnp.int32),
            pltpu.SemaphoreType.DMA,
        ],
    )
    def sc_router(logits_hbm, scores_hbm, idx_hbm, lg_v, sc_v, ix_v, sem):
        wid = lax.axis_index("s") * nc + lax.axis_index("c")
        base = wid * rows_per_w
        pltpu.sync_copy(logits_hbm.at[pl.ds(base, rows_per_w)], lg_v)

        def row_body(r, _):
            vs = [lg_v[r, pl.ds(v * nl, nl)] for v in range(nv)]
            neg_inf = jnp.float32(-jnp.inf)
            work = vs
            ms = []
            for _k in range(_K):
                m4 = jnp.maximum(jnp.maximum(work[0], work[1]),
                                 jnp.maximum(work[2], work[3]))
                m = _bfly(m4, jnp.maximum)
                work = [jnp.where(w == m, neg_inf, w) for w in work]
                ms.append(m)
            m0 = ms[0]
            es = []
            acc = None
            for v in range(nv):
                selv = work[v] == neg_inf
                ev = jnp.where(selv, jnp.exp(vs[v] - m0), 0.0)
                acc = ev if acc is None else acc + ev
                es.append(ev)
            tot = _bfly(acc, jnp.add)
            for v in range(nv):
                sc_v[r, pl.ds(v * nl, nl)] = es[v] / tot
            a1 = None
            a2 = None
            for v in range(nv):
                iota = lax.iota(jnp.int32, nl) + v * nl
                rank = jnp.zeros((nl,), jnp.int32)
                for k in range(1, _K):
                    rank = rank + jnp.where(vs[v] < ms[k - 1], 1, 0)
                selv = work[v] == neg_inf
                sub = jnp.where(rank >= 4, rank - 4, rank)
                contrib = jnp.where(selv, iota << (6 * (3 - sub)), 0)
                lo = rank < 4
                c1 = jnp.where(lo, contrib, 0)
                c2 = contrib - c1
                a1 = c1 if a1 is None else a1 + c1
                a2 = c2 if a2 is None else a2 + c2
            w1 = _bfly(a1, jnp.add)
            w2 = _bfly(a2, jnp.add)
            kio = lax.iota(jnp.int32, nl)
            kv = jnp.where(kio < 4, w1, w2)
            ksub = jnp.minimum(jnp.where(kio >= 4, kio - 4, kio), 3)
            idx16 = (kv >> (6 * (3 - ksub))) & (_EXPERTS - 1)
            ix_v[r, :] = idx16
            return _

        lax.fori_loop(0, rows_per_w, row_body, 0)
        pltpu.sync_copy(sc_v, scores_hbm.at[pl.ds(base, rows_per_w)])
        pltpu.sync_copy(ix_v, idx_hbm.at[pl.ds(base, rows_per_w)])

    return sc_router


@jax.jit
def kernel(hidden_states, weight, bias):
    x = hidden_states.reshape(-1, _HIDDEN)
    wt = weight.T
    b = bias.reshape(1, _EXPERTS)
    logits = _tc_logits(x, wt, b)
    scores, idx16 = _make_sc_router()(logits)
    return (scores, idx16[:, :_K])


# speedup vs baseline: 1.3283x; 1.3283x over previous
import jax
import jax.numpy as jnp
from jax import lax
from jax.experimental import pallas as pl
from jax.experimental.pallas import tpu as pltpu

_TOKENS = 8192
_HIDDEN = 2048
_EXPERTS = 64
_K = 8
_BT = 1024
_KC = 2
_HC = _HIDDEN // _KC


def _router_body(x_ref, wt_ref, b_ref, scores_ref, idx_ref):
    kstep = pl.program_id(1)
    partial = jnp.dot(x_ref[...], wt_ref[...],
                      preferred_element_type=jnp.float32)

    @pl.when(kstep == 0)
    def _first():
        scores_ref[...] = partial + b_ref[...]

    @pl.when(kstep == _KC - 1)
    def _last():
        logits = scores_ref[...] + partial

        neg_inf = jnp.float32(-jnp.inf)
        work = logits
        m_cols = []
        for _ in range(_K):
            m = jnp.max(work, axis=1, keepdims=True)
            work = jnp.where(work == m, neg_inf, work)
            m_cols.append(m)

        sel_mask = work == neg_inf
        e = jnp.where(sel_mask, jnp.exp(logits - m_cols[0]), 0.0)
        s = jnp.sum(e, axis=1, keepdims=True)
        scores_ref[...] = e / s

        b2 = logits < m_cols[3]
        b1 = logits < jnp.where(b2, m_cols[5], m_cols[1])
        p2a = jnp.where(b1, m_cols[2], m_cols[0])
        p2b = jnp.where(b1, m_cols[6], m_cols[4])
        b0 = logits < jnp.where(b2, p2b, p2a)

        iota = lax.broadcasted_iota(jnp.int32, logits.shape, 1)
        sub = jnp.where(b1, 2, 0) + jnp.where(b0, 1, 0)
        contrib = jnp.where(sel_mask, iota << (18 - 6 * sub), 0)
        c1 = jnp.where(b2, 0, contrib)
        w1 = jnp.sum(c1.astype(jnp.float32), axis=1, keepdims=True)
        w2 = jnp.sum((contrib - c1).astype(jnp.float32), axis=1,
                     keepdims=True)

        w = jnp.concatenate([w1] * 4 + [w2] * 4, axis=1).astype(jnp.int32)
        kio = lax.broadcasted_iota(jnp.int32, (logits.shape[0], _K), 1)
        ksub = jnp.where(kio >= 4, kio - 4, kio)
        idx_ref[...] = (w >> (6 * (3 - ksub))) & (_EXPERTS - 1)


@jax.jit
def kernel(hidden_states, weight, bias):
    x = hidden_states.reshape(-1, _HIDDEN)
    wt = weight.T
    b = bias.reshape(1, _EXPERTS)
    grid = (_TOKENS // _BT, _KC)
    scores, idx = pl.pallas_call(
        _router_body,
        grid=grid,
        in_specs=[
            pl.BlockSpec((_BT, _HC), lambda i, k: (i, k)),
            pl.BlockSpec((_HC, _EXPERTS), lambda i, k: (k, 0)),
            pl.BlockSpec((1, _EXPERTS), lambda i, k: (0, 0)),
        ],
        out_specs=[
            pl.BlockSpec((_BT, _EXPERTS), lambda i, k: (i, 0)),
            pl.BlockSpec((_BT, _K), lambda i, k: (i, 0)),
        ],
        out_shape=[
            jax.ShapeDtypeStruct((_TOKENS, _EXPERTS), jnp.float32),
            jax.ShapeDtypeStruct((_TOKENS, _K), jnp.int32),
        ],
        compiler_params=pltpu.CompilerParams(
            dimension_semantics=("arbitrary", "arbitrary"),
        ),
    )(x, wt, b)
    return (scores, idx)
